# baseline (speedup 1.0000x reference)
import jax
import jax.numpy as jnp
from jax import lax
from jax.experimental import pallas as pl

CH = 8192
KK = 512
W = 1024
R = 128
CHUNK = 5


def _popcnt16(x):
    x = x - ((x >> 1) & 0x5555)
    x = (x & 0x3333) + ((x >> 2) & 0x3333)
    x = (x + (x >> 4)) & 0x0F0F
    return (x + (x >> 8)) & 0x1F


def _pack_body(bits_ref, out_ref):
    b = bits_ref[...]
    sh = lax.broadcasted_iota(jnp.int32, b.shape, 1)
    out_ref[...] = jnp.sum(b << sh, axis=1, keepdims=True).astype(jnp.bfloat16)


def _main_body(class_ref, compo_ref, coprob_ref, packed_ref,
               cidx_ref, cscore_ref, cs_ref, adj_ref,
               hidx_ref, hscore_ref, p1_ref, p2_ref, p3_ref):
    iota_c = lax.broadcasted_iota(jnp.int32, (R, CH), 1)
    iota_k = lax.broadcasted_iota(jnp.int32, (R, KK), 1)
    iota_w = lax.broadcasted_iota(jnp.int32, (R, W), 1)
    packed = packed_ref[...]

    x = class_ref[...]
    m = jnp.max(x, axis=1, keepdims=True)
    e = jnp.exp(x - m)
    s = jnp.sum(e, axis=1, keepdims=True)
    pm = e
    idx_l, val_l = [], []
    for _ in range(10):
        cur = jnp.max(pm, axis=1, keepdims=True)
        idx = jnp.min(jnp.where(pm == cur, iota_c, CH), axis=1, keepdims=True)
        idx_l.append(idx)
        val_l.append(cur / s)
        pm = jnp.where(iota_c == idx, -1.0, pm)
    class_indices = jnp.concatenate(idx_l, axis=1)
    class_scores = jnp.concatenate(val_l, axis=1)

    cs = jax.nn.sigmoid(compo_ref[...])
    w = jnp.where(cs > 0.88, cs, 0.0)
    adj = jnp.dot(w, coprob_ref[...],
                  preferred_element_type=jnp.float32) * 0.1 + cs
    amax = jnp.max(adj, axis=1, keepdims=True)
    max_idx = jnp.min(jnp.where(adj == amax, iota_k, KK), axis=1, keepdims=True)
    onehot0 = (iota_k == max_idx).astype(jnp.float32)
    max_score = jnp.sum(onehot0 * cs, axis=1, keepdims=True)
    ns = jnp.sum((cs > 0.7).astype(jnp.int32), axis=1, keepdims=True)
    hit0 = jnp.dot(onehot0.astype(jnp.bfloat16), packed,
                   preferred_element_type=jnp.float32).astype(jnp.int32)

    def nbits2(h):
        t = h & (h - 1)
        z = (h != 0).astype(jnp.int32) + (t != 0).astype(jnp.int32)
        return jnp.sum(z, axis=1, keepdims=True)

    s0 = jnp.max(cs, axis=1, keepdims=True)
    o0 = jnp.min(jnp.where(cs == s0, iota_k, KK), axis=1, keepdims=True)
    csm0 = jnp.where(iota_k == o0, -1.0, cs)
    i0 = jnp.ones((R, 1), jnp.int32)

    def cond(st):
        i, hit, prev, csm, last, second = st
        return jnp.any((i < ns) & (nbits2(hit) > 1))

    def body(st):
        i, hit, prev, csm, last, second = st
        for _ in range(CHUNK):
            act = (i < ns) & (nbits2(hit) > 1)
            cur = jnp.max(csm, axis=1, keepdims=True)
            cidx = jnp.min(jnp.where(csm == cur, iota_k, KK),
                           axis=1, keepdims=True)
            sel = iota_k == cidx
            csm = jnp.where(sel, -1.0, csm)
            row = jnp.dot(sel.astype(jnp.bfloat16), packed,
                          preferred_element_type=jnp.float32).astype(jnp.int32)
            prev = jnp.where(act, hit, prev)
            hit = jnp.where(act, hit & row, hit)
            i = i + act.astype(jnp.int32)
            second = jnp.where(act, last, second)
            last = jnp.where(act, cur, last)
        return (i, hit, prev, csm, last, second)

    i, hit, prev, csm, last, second = lax.while_loop(
        cond, body, (i0, hit0, hit0, csm0, s0, s0))

    revert = nbits2(hit) == 0
    hit_final = jnp.where(revert, prev, hit)
    i_out = i - revert.astype(jnp.int32)
    sel_s = jnp.where(revert, second, last)
    hit_score = jnp.where(i_out == 1, max_score, sel_s)

    hm = hit_final
    hidx_l = []
    for _ in range(10):
        wstar = jnp.min(jnp.where(hm != 0, iota_w, W), axis=1, keepdims=True)
        selw = iota_w == wstar
        wval = jnp.sum(jnp.where(selw, hm, 0), axis=1, keepdims=True)
        lsb = wval & (-wval)
        bidx = _popcnt16(lsb - 1)
        fidx = jnp.where(wstar == W, CH, wstar * 8 + bidx)
        hidx_l.append(fidx)
        hm = jnp.where(selw, hm & (hm - 1), hm)
    h10 = jnp.concatenate(hidx_l, axis=1)
    hit_indices = jnp.where(h10 == CH, -1, h10)

    nch = jnp.sum((hit_indices != -1).astype(jnp.int32), axis=1, keepdims=True)
    ci0 = class_indices[:, 0:1]
    hi0 = hit_indices[:, 0:1]
    cp1 = jnp.where(nch == 1, hi0, ci0)
    cp2 = jnp.where((class_scores[:, 0:1] < 0.85) & (nch == 1), hi0, ci0)
    match = class_indices == hit_indices[:, 0:1]
    for k in range(1, 10):
        match = match | (class_indices == hit_indices[:, k:k + 1])
    iota10 = lax.broadcasted_iota(jnp.int32, (R, 10), 1)
    jstar = jnp.min(jnp.where(match, iota10, 10), axis=1, keepdims=True)
    jstar = jnp.where(jstar == 10, 0, jstar)
    cp3 = jnp.sum(jnp.where(iota10 == jstar, class_indices, 0),
                  axis=1, keepdims=True)

    cidx_ref[...] = class_indices
    cscore_ref[...] = class_scores
    cs_ref[...] = cs
    adj_ref[...] = adj
    hidx_ref[...] = hit_indices
    hscore_ref[...] = hit_score
    p1_ref[...] = cp1
    p2_ref[...] = cp2
    p3_ref[...] = cp3


def _pack(mat, interpret=False):
    bits = mat.reshape(KK * W, 8)
    nrows = KK * W
    blk = 8192
    packed = pl.pallas_call(
        _pack_body,
        grid=(nrows // blk,),
        in_specs=[pl.BlockSpec((blk, 8), lambda i: (i, 0))],
        out_specs=pl.BlockSpec((blk, 1), lambda i: (i, 0)),
        out_shape=jax.ShapeDtypeStruct((nrows, 1), jnp.bfloat16),
        interpret=interpret,
    )(bits)
    return packed.reshape(KK, W)


def _run(pred_class_logits, pred_compo_logits, coprob, mat, interpret=False):
    B = pred_class_logits.shape[0]
    packed = _pack(mat, interpret=interpret)
    outs = pl.pallas_call(
        _main_body,
        grid=(B // R,),
        in_specs=[
            pl.BlockSpec((R, CH), lambda i: (i, 0)),
            pl.BlockSpec((R, KK), lambda i: (i, 0)),
            pl.BlockSpec((KK, KK), lambda i: (0, 0)),
            pl.BlockSpec((KK, W), lambda i: (0, 0)),
        ],
        out_specs=[
            pl.BlockSpec((R, 10), lambda i: (i, 0)),
            pl.BlockSpec((R, 10), lambda i: (i, 0)),
            pl.BlockSpec((R, KK), lambda i: (i, 0)),
            pl.BlockSpec((R, KK), lambda i: (i, 0)),
            pl.BlockSpec((R, 10), lambda i: (i, 0)),
            pl.BlockSpec((R, 1), lambda i: (i, 0)),
            pl.BlockSpec((R, 1), lambda i: (i, 0)),
            pl.BlockSpec((R, 1), lambda i: (i, 0)),
            pl.BlockSpec((R, 1), lambda i: (i, 0)),
        ],
        out_shape=[
            jax.ShapeDtypeStruct((B, 10), jnp.int32),
            jax.ShapeDtypeStruct((B, 10), jnp.float32),
            jax.ShapeDtypeStruct((B, KK), jnp.float32),
            jax.ShapeDtypeStruct((B, KK), jnp.float32),
            jax.ShapeDtypeStruct((B, 10), jnp.int32),
            jax.ShapeDtypeStruct((B, 1), jnp.float32),
            jax.ShapeDtypeStruct((B, 1), jnp.int32),
            jax.ShapeDtypeStruct((B, 1), jnp.int32),
            jax.ShapeDtypeStruct((B, 1), jnp.int32),
        ],
        interpret=interpret,
    )(pred_class_logits, pred_compo_logits, coprob, packed)
    cidx, cscore, cs, adj, hidx, hscore, p1, p2, p3 = outs
    return (cidx, cscore, cs, adj, hidx, hscore[:, 0], p1[:, 0], p2[:, 0], p3[:, 0])


def kernel(pred_class_logits, pred_compo_logits, chinese_char_ids,
           compo_co_occurrence_prob, compo_chinese_matrix):
    return _run(pred_class_logits, pred_compo_logits,
                compo_co_occurrence_prob, compo_chinese_matrix)


# speedup vs baseline: 2.3480x; 2.3480x over previous
import jax
import jax.numpy as jnp
from jax import lax
from jax.experimental import pallas as pl

CH = 8192
KK = 512
W = 1024
R = 128
CHUNK = 8


def _popcnt16(x):
    x = x - ((x >> 1) & 0x5555)
    x = (x & 0x3333) + ((x >> 2) & 0x3333)
    x = (x + (x >> 4)) & 0x0F0F
    return (x + (x >> 8)) & 0x1F


def _pack_body(bits_ref, out_ref):
    b = bits_ref[...]
    li = lax.broadcasted_iota(jnp.int32, (W, 128), 0)
    jj = lax.broadcasted_iota(jnp.int32, (W, 128), 1)
    wt = jnp.where((li >> 3) == jj, 1 << (li & 7), 0).astype(jnp.bfloat16)
    out_ref[...] = jnp.dot(b, wt,
                           preferred_element_type=jnp.float32).astype(jnp.bfloat16)


def _main_body(class_ref, compo_ref, coprob_ref, packed_ref,
               cidx_ref, cscore_ref, cs_ref, adj_ref,
               hidx_ref, hscore_ref, p1_ref, p2_ref, p3_ref):
    iota_c = lax.broadcasted_iota(jnp.int32, (R, CH), 1)
    iota_k = lax.broadcasted_iota(jnp.int32, (R, KK), 1)
    iota_w = lax.broadcasted_iota(jnp.int32, (R, W), 1)
    packed = packed_ref[...]

    x = class_ref[...]
    m = jnp.max(x, axis=1, keepdims=True)
    e = jnp.exp(x - m)
    s = jnp.sum(e, axis=1, keepdims=True)
    pm = e
    idx_l, val_l = [], []
    for _ in range(10):
        cur = jnp.max(pm, axis=1, keepdims=True)
        idx = jnp.min(jnp.where(pm == cur, iota_c, CH), axis=1, keepdims=True)
        idx_l.append(idx)
        val_l.append(cur / s)
        pm = jnp.where(iota_c == idx, -1.0, pm)
    class_indices = jnp.concatenate(idx_l, axis=1)
    class_scores = jnp.concatenate(val_l, axis=1)

    cs = jax.nn.sigmoid(compo_ref[...])
    w = jnp.where(cs > 0.88, cs, 0.0)
    adj = jnp.dot(w, coprob_ref[...],
                  preferred_element_type=jnp.float32) * 0.1 + cs
    amax = jnp.max(adj, axis=1, keepdims=True)
    max_idx = jnp.min(jnp.where(adj == amax, iota_k, KK), axis=1, keepdims=True)
    onehot0 = (iota_k == max_idx).astype(jnp.float32)
    max_score = jnp.sum(onehot0 * cs, axis=1, keepdims=True)
    ns = jnp.sum((cs > 0.7).astype(jnp.int32), axis=1, keepdims=True)
    hit0 = jnp.dot(onehot0.astype(jnp.bfloat16), packed,
                   preferred_element_type=jnp.float32).astype(jnp.int32)

    def nbits2(h):
        t = h & (h - 1)
        z = (h != 0).astype(jnp.int32) + (t != 0).astype(jnp.int32)
        return jnp.sum(z, axis=1, keepdims=True)

    s0 = jnp.max(cs, axis=1, keepdims=True)
    o0 = jnp.min(jnp.where(cs == s0, iota_k, KK), axis=1, keepdims=True)
    csm0 = jnp.where(iota_k == o0, -1.0, cs)
    i0 = jnp.ones((R, 1), jnp.int32)

    def cond(st):
        i, hit, prev, csm, last, second = st
        return jnp.any((i < ns) & (nbits2(hit) > 1))

    def body(st):
        i, hit, prev, csm, last, second = st
        for _ in range(CHUNK):
            act = (i < ns) & (nbits2(hit) > 1)
            cur = jnp.max(csm, axis=1, keepdims=True)
            cidx = jnp.min(jnp.where(csm == cur, iota_k, KK),
                           axis=1, keepdims=True)
            sel = iota_k == cidx
            csm = jnp.where(sel, -1.0, csm)
            row = jnp.dot(sel.astype(jnp.bfloat16), packed,
                          preferred_element_type=jnp.float32).astype(jnp.int32)
            prev = jnp.where(act, hit, prev)
            hit = jnp.where(act, hit & row, hit)
            i = i + act.astype(jnp.int32)
            second = jnp.where(act, last, second)
            last = jnp.where(act, cur, last)
        return (i, hit, prev, csm, last, second)

    i, hit, prev, csm, last, second = lax.while_loop(
        cond, body, (i0, hit0, hit0, csm0, s0, s0))

    revert = nbits2(hit) == 0
    hit_final = jnp.where(revert, prev, hit)
    i_out = i - revert.astype(jnp.int32)
    sel_s = jnp.where(revert, second, last)
    hit_score = jnp.where(i_out == 1, max_score, sel_s)

    hm = hit_final
    hidx_l = []
    for _ in range(10):
        wstar = jnp.min(jnp.where(hm != 0, iota_w, W), axis=1, keepdims=True)
        selw = iota_w == wstar
        wval = jnp.sum(jnp.where(selw, hm, 0), axis=1, keepdims=True)
        lsb = wval & (-wval)
        bidx = _popcnt16(lsb - 1)
        fidx = jnp.where(wstar == W, CH, wstar * 8 + bidx)
        hidx_l.append(fidx)
        hm = jnp.where(selw, hm & (hm - 1), hm)
    h10 = jnp.concatenate(hidx_l, axis=1)
    hit_indices = jnp.where(h10 == CH, -1, h10)

    nch = jnp.sum((hit_indices != -1).astype(jnp.int32), axis=1, keepdims=True)
    ci0 = class_indices[:, 0:1]
    hi0 = hit_indices[:, 0:1]
    cp1 = jnp.where(nch == 1, hi0, ci0)
    cp2 = jnp.where((class_scores[:, 0:1] < 0.85) & (nch == 1), hi0, ci0)
    match = class_indices == hit_indices[:, 0:1]
    for k in range(1, 10):
        match = match | (class_indices == hit_indices[:, k:k + 1])
    iota10 = lax.broadcasted_iota(jnp.int32, (R, 10), 1)
    jstar = jnp.min(jnp.where(match, iota10, 10), axis=1, keepdims=True)
    jstar = jnp.where(jstar == 10, 0, jstar)
    cp3 = jnp.sum(jnp.where(iota10 == jstar, class_indices, 0),
                  axis=1, keepdims=True)

    cidx_ref[...] = class_indices
    cscore_ref[...] = class_scores
    cs_ref[...] = cs
    adj_ref[...] = adj
    hidx_ref[...] = hit_indices
    hscore_ref[...] = hit_score
    p1_ref[...] = cp1
    p2_ref[...] = cp2
    p3_ref[...] = cp3


def _pack(mat, interpret=False):
    bits = mat.reshape(KK * 8, W).astype(jnp.bfloat16)
    packed = pl.pallas_call(
        _pack_body,
        grid=(1,),
        in_specs=[pl.BlockSpec((KK * 8, W), lambda i: (0, 0))],
        out_specs=pl.BlockSpec((KK * 8, 128), lambda i: (0, 0)),
        out_shape=jax.ShapeDtypeStruct((KK * 8, 128), jnp.bfloat16),
        interpret=interpret,
    )(bits)
    return packed.reshape(KK, W)


def _run(pred_class_logits, pred_compo_logits, coprob, mat, interpret=False):
    B = pred_class_logits.shape[0]
    packed = _pack(mat, interpret=interpret)
    outs = pl.pallas_call(
        _main_body,
        grid=(B // R,),
        in_specs=[
            pl.BlockSpec((R, CH), lambda i: (i, 0)),
            pl.BlockSpec((R, KK), lambda i: (i, 0)),
            pl.BlockSpec((KK, KK), lambda i: (0, 0)),
            pl.BlockSpec((KK, W), lambda i: (0, 0)),
        ],
        out_specs=[
            pl.BlockSpec((R, 10), lambda i: (i, 0)),
            pl.BlockSpec((R, 10), lambda i: (i, 0)),
            pl.BlockSpec((R, KK), lambda i: (i, 0)),
            pl.BlockSpec((R, KK), lambda i: (i, 0)),
            pl.BlockSpec((R, 10), lambda i: (i, 0)),
            pl.BlockSpec((R, 1), lambda i: (i, 0)),
            pl.BlockSpec((R, 1), lambda i: (i, 0)),
            pl.BlockSpec((R, 1), lambda i: (i, 0)),
            pl.BlockSpec((R, 1), lambda i: (i, 0)),
        ],
        out_shape=[
            jax.ShapeDtypeStruct((B, 10), jnp.int32),
            jax.ShapeDtypeStruct((B, 10), jnp.float32),
            jax.ShapeDtypeStruct((B, KK), jnp.float32),
            jax.ShapeDtypeStruct((B, KK), jnp.float32),
            jax.ShapeDtypeStruct((B, 10), jnp.int32),
            jax.ShapeDtypeStruct((B, 1), jnp.float32),
            jax.ShapeDtypeStruct((B, 1), jnp.int32),
            jax.ShapeDtypeStruct((B, 1), jnp.int32),
            jax.ShapeDtypeStruct((B, 1), jnp.int32),
        ],
        interpret=interpret,
    )(pred_class_logits, pred_compo_logits, coprob, packed)
    cidx, cscore, cs, adj, hidx, hscore, p1, p2, p3 = outs
    return (cidx, cscore, cs, adj, hidx, hscore[:, 0], p1[:, 0], p2[:, 0], p3[:, 0])


def kernel(pred_class_logits, pred_compo_logits, chinese_char_ids,
           compo_co_occurrence_prob, compo_chinese_matrix):
    return _run(pred_class_logits, pred_compo_logits,
                compo_co_occurrence_prob, compo_chinese_matrix)
